# baseline (speedup 1.0000x reference)
import functools

import jax
import jax.numpy as jnp
from jax import lax
from jax.experimental import pallas as pl
from jax.experimental.pallas import tpu as pltpu
from jax.experimental.pallas import tpu_sc as plsc

N = 10000
E = 640000
NPAD = 10112
NC, NS = 2, 16
NW = NC * NS
EB = 112
BLK_C0 = 208
BLK_C1 = 152
NBLK = NS * (BLK_C0 + BLK_C1)
EPAD = NBLK * EB
RBLK = 1000
GRID = N // RBLK


def _make_sc_edge_kernel(RW, H):
    RWE = RW + 16
    rows_per_tile = NPAD // NS
    nvec = EB // 16
    mesh = plsc.VectorSubcoreMesh(core_axis_name="c", subcore_axis_name="s")

    @functools.partial(
        pl.kernel,
        out_type=(
            jax.ShapeDtypeStruct((NPAD, RWE), jnp.float32),
            jax.ShapeDtypeStruct((NPAD, RWE), jnp.float32),
        ),
        mesh=mesh,
        compiler_params=pltpu.CompilerParams(
            needs_layout_passes=False, use_tc_tiling_on_sc=False),
        scratch_types=[
            pltpu.VMEM((2, EB), jnp.int32),
            pltpu.VMEM((4, EB), jnp.int32),
            pltpu.VMEM((EB, RWE), jnp.float32),
            pltpu.VMEM((EB, RWE), jnp.float32),
            pltpu.VMEM((2, EB, 16), jnp.float32),
            pltpu.VMEM_SHARED((NPAD, RWE), jnp.float32),
        ] + [pltpu.SemaphoreType.DMA] * 8,
    )
    def sc_kernel(src_hbm, dst_hbm, g_hbm, ad_hbm,
                  acc0, acc1,
                  sbuf, dbuf, rows0, rows1, adb, acc_sp,
                  is0, is1, id0, id1, gs0, gs1, ss0, ss1):
        cid = lax.axis_index("c")
        sid = lax.axis_index("s")
        nb = jnp.where(cid == 0, BLK_C0, BLK_C1)
        rows = (rows0, rows1)
        isems = (is0, is1)
        idsems = (id0, id1)
        gsems = (gs0, gs1)
        ssems = (ss0, ss1)

        zero16 = jnp.zeros((16,), jnp.float32)

        @pl.loop(0, EB)
        def _zero_rows(e):
            for k in range(RWE // 16):
                rows0[e, pl.ds(k * 16, 16)] = zero16

        base = sid * rows_per_tile
        nfull = rows_per_tile // EB
        rem = rows_per_tile - nfull * EB
        for ch in range(nfull):
            pltpu.sync_copy(rows0, acc_sp.at[pl.ds(base + ch * EB, EB)])
        if rem:
            pltpu.sync_copy(rows0.at[pl.ds(0, rem)],
                            acc_sp.at[pl.ds(base + nfull * EB, rem)])
        plsc.subcore_barrier()

        iot = lax.iota(jnp.int32, 16)
        base_row = jnp.where(cid == 0, sid * BLK_C0,
                             NS * BLK_C0 + sid * BLK_C1)

        def compute_block(rws, adbj):
            @plsc.parallel_loop(0, EB, step=16, unroll=nvec)
            def _fphase(v16):
                ev16 = v16 + iot
                for h in range(H):
                    asv = plsc.load_gather(
                        rws, [ev16, jnp.full((16,), RW + h, jnp.int32)])
                    adv = plsc.load_gather(
                        adbj, [ev16, jnp.full((16,), h, jnp.int32)])
                    ev = asv + adv
                    ev = jnp.where(ev > 0, ev, ev * 0.2)
                    fv = jnp.exp(ev)
                    plsc.store_scatter(
                        rws, [ev16, jnp.full((16,), RW + h, jnp.int32)], fv)

            @plsc.parallel_loop(0, EB, unroll=4)
            def _scale(e):
                erow = jnp.full((16,), e, jnp.int32)
                for h in range(H):
                    fvec = plsc.load_gather(
                        rws, [erow, jnp.full((16,), RW + h, jnp.int32)])
                    for k in range(2):
                        col = h * 32 + k * 16
                        rws[e, pl.ds(col, 16)] = rws[e, pl.ds(col, 16)] * fvec

        pltpu.sync_copy(src_hbm.at[base_row], sbuf.at[0])
        pltpu.sync_copy(dst_hbm.at[base_row], dbuf.at[0])
        pltpu.async_copy(g_hbm.at[sbuf.at[0]], rows0, gs0)
        pltpu.async_copy(ad_hbm.at[dbuf.at[0]], adb.at[0], is0)

        @pl.loop(0, nb, step=4)
        def _outer(b0):
            for u in range(4):
                b = b0 + u
                j = u % 2
                nj = (u + 1) % 2
                nd = (u + 1) % 4

                @pl.when(b + 1 < nb)
                def _issue_idx():
                    pltpu.async_copy(src_hbm.at[base_row + b + 1],
                                     sbuf.at[nj], isems[nj])
                    pltpu.async_copy(dst_hbm.at[base_row + b + 1],
                                     dbuf.at[nd], idsems[nj])

                @pl.when(b >= 1)
                def _wait_scatter():
                    pltpu.make_async_copy(
                        rows[nj], acc_sp.at[dbuf.at[(u + 3) % 4]],
                        ssems[nj]).wait()

                @pl.when(b + 1 < nb)
                def _issue_gather():
                    pltpu.make_async_copy(src_hbm.at[base_row + b + 1],
                                          sbuf.at[nj], isems[nj]).wait()
                    pltpu.make_async_copy(dst_hbm.at[base_row + b + 1],
                                          dbuf.at[nd], idsems[nj]).wait()
                    pltpu.async_copy(g_hbm.at[sbuf.at[nj]], rows[nj],
                                     gsems[nj])
                    pltpu.async_copy(ad_hbm.at[dbuf.at[nd]], adb.at[nj],
                                     isems[nj])

                pltpu.make_async_copy(g_hbm.at[sbuf.at[j]], rows[j],
                                      gsems[j]).wait()
                pltpu.make_async_copy(ad_hbm.at[dbuf.at[u]], adb.at[j],
                                      isems[j]).wait()
                compute_block(rows[j], adb.at[j])
                pltpu.async_copy(rows[j], acc_sp.at[dbuf.at[u]], ssems[j],
                                 add=True)

        pltpu.make_async_copy(rows[1], acc_sp.at[dbuf.at[3]], ssems[1]).wait()
        plsc.subcore_barrier()

        @pl.when(cid == 0)
        def _out0():
            pltpu.sync_copy(acc_sp.at[pl.ds(base, rows_per_tile)],
                            acc0.at[pl.ds(base, rows_per_tile)])

        @pl.when(cid == 1)
        def _out1():
            pltpu.sync_copy(acc_sp.at[pl.ds(base, rows_per_tile)],
                            acc1.at[pl.ds(base, rows_per_tile)])

    return sc_kernel


_sc_edge_128 = _make_sc_edge_kernel(128, 4)
_sc_edge_32 = _make_sc_edge_kernel(32, 1)


def _full(shape):
    return pl.BlockSpec(shape, lambda i: (0,) * len(shape))


def _rows(width):
    return pl.BlockSpec((RBLK, width), lambda i: (i, 0))


def _logit_tables(xw, asf, adf, st, H):
    asn = jnp.dot(xw * asf, st, preferred_element_type=jnp.float32,
                  precision=lax.Precision.HIGHEST)
    adn = jnp.dot(xw * adf, st, preferred_element_type=jnp.float32,
                  precision=lax.Precision.HIGHEST)
    z = jnp.zeros((xw.shape[0], 16 - H), jnp.float32)
    g = jnp.concatenate([xw, asn, z], axis=1)
    ad16 = jnp.concatenate([adn, z], axis=1)
    return g, ad16


def _tc_pre_body(H, x_ref, w_ref, asf_ref, adf_ref, st_ref, g_ref, ad_ref):
    xw = jnp.dot(x_ref[...], w_ref[...], preferred_element_type=jnp.float32)
    g, ad16 = _logit_tables(xw, asf_ref[...], adf_ref[...], st_ref[...], H)
    g_ref[...] = g
    ad_ref[...] = ad16


def _tc_pre(x, w, asf, adf, st, H, RW):
    return pl.pallas_call(
        functools.partial(_tc_pre_body, H),
        grid=(GRID,),
        in_specs=[_rows(x.shape[1]), _full(w.shape), _full(asf.shape),
                  _full(adf.shape), _full(st.shape)],
        out_specs=[_rows(RW + 16), _rows(16)],
        out_shape=[
            jax.ShapeDtypeStruct((N, RW + 16), jnp.float32),
            jax.ShapeDtypeStruct((N, 16), jnp.float32),
        ],
    )(x, w, asf, adf, st)


def _elu(g):
    return jnp.where(g > 0, g, jnp.exp(g) - 1.0)


def _gat_epilogue(acc0, acc1, g, ad16, br, s, H, RW):
    xw = g[:, :RW]
    ev = g[:, RW:RW + H] + ad16[:, :H]
    fs = jnp.exp(jnp.where(ev > 0, ev, 0.2 * ev))
    dtot = acc0[:, RW:RW + H] + acc1[:, RW:RW + H] + fs
    num = acc0[:, :RW] + acc1[:, :RW] + jnp.dot(
        fs, s, preferred_element_type=jnp.float32,
        precision=lax.Precision.HIGHEST) * xw
    den = jnp.dot(dtot, s, preferred_element_type=jnp.float32,
                  precision=lax.Precision.HIGHEST) + 1e-16
    return _elu(num / den + br)


def _make_combine_mid(H, RW, residual):
    def body(*refs):
        if residual:
            (acc0, acc1, gr, adr, br, sr, resr,
             wn, asfn, adfn, stn, HN, h_out, gn_out, adn_out) = refs
        else:
            (acc0, acc1, gr, adr, br, sr,
             wn, asfn, adfn, stn, HN, h_out, gn_out, adn_out) = refs
        h = _gat_epilogue(acc0[...], acc1[...], gr[...], adr[...],
                          br[...], sr[...], H, RW)
        if residual:
            h = h + resr[...]
        h_out[...] = h
        xwn = jnp.dot(h, wn[...], preferred_element_type=jnp.float32)
        gn, adn16 = _logit_tables(xwn, asfn[...], adfn[...], stn[...], HN)
        gn_out[...] = gn
        adn_out[...] = adn16

    def call(acc0, acc1, g, ad16, b, s, res, wn, asfn, adfn, stn, HN, RWN):
        ins = [acc0, acc1, g, ad16, b, s]
        specs = [_rows(RW + 16), _rows(RW + 16), _rows(RW + 16), _rows(16),
                 _full(b.shape), _full(s.shape)]
        if residual:
            ins.append(res)
            specs.append(_rows(RW))
        ins += [wn, asfn, adfn, stn]
        specs += [_full(wn.shape), _full(asfn.shape), _full(adfn.shape),
                  _full(stn.shape)]
        return pl.pallas_call(
            lambda *refs: body(*refs[:len(ins)], HN, *refs[len(ins):]),
            grid=(GRID,),
            in_specs=specs,
            out_specs=[_rows(RW), _rows(RWN + 16), _rows(16)],
            out_shape=[
                jax.ShapeDtypeStruct((N, RW), jnp.float32),
                jax.ShapeDtypeStruct((N, RWN + 16), jnp.float32),
                jax.ShapeDtypeStruct((N, 16), jnp.float32),
            ],
        )(*ins)

    return call


_combine1 = _make_combine_mid(4, 128, residual=False)
_combine2 = _make_combine_mid(4, 128, residual=True)


def _combine_final_body(acc0, acc1, gr, adr, br, s1, wrc, brc, rc_out):
    h = _gat_epilogue(acc0[...], acc1[...], gr[...], adr[...],
                      br[...], s1[...], 1, 32)
    rc_out[...] = jnp.dot(h, wrc[...],
                          preferred_element_type=jnp.float32) + brc[...]


def _combine_final(acc0, acc1, g, ad16, b, s1, wrc, brc):
    return pl.pallas_call(
        _combine_final_body,
        grid=(GRID,),
        in_specs=[_rows(48), _rows(48), _rows(48), _rows(16),
                  _full(b.shape), _full(s1.shape), _full(wrc.shape),
                  _full(brc.shape)],
        out_specs=_rows(2),
        out_shape=jax.ShapeDtypeStruct((N, 2), jnp.float32),
    )(acc0, acc1, g, ad16, b, s1, wrc, brc)


def kernel(x, edge_index, W1, a_src1, a_dst1, b1, W2, a_src2, a_dst2, b2,
           W3, a_src3, a_dst3, b3, Wr, br, Wc, bc):
    pad = EPAD - E
    srcp = jnp.concatenate(
        [edge_index[0], jnp.zeros((pad,), jnp.int32)]).reshape(-1, EB)
    dstp = jnp.concatenate(
        [edge_index[1], jnp.full((pad,), N, jnp.int32)]).reshape(-1, EB)

    s4 = jnp.repeat(jnp.eye(4, dtype=jnp.float32), 32, axis=1)
    st4 = s4.T
    s1 = jnp.ones((1, 32), jnp.float32)
    st1 = jnp.ones((32, 1), jnp.float32)

    asf1 = a_src1.reshape(1, 128)
    adf1 = a_dst1.reshape(1, 128)
    asf2 = a_src2.reshape(1, 128)
    adf2 = a_dst2.reshape(1, 128)
    asf3 = a_src3.reshape(1, 32)
    adf3 = a_dst3.reshape(1, 32)
    b1r = b1.reshape(1, 128)
    b2r = b2.reshape(1, 128)
    b3r = b3.reshape(1, 32)
    wrc = jnp.concatenate([Wr, Wc], axis=1)
    brc = jnp.concatenate([br, bc]).reshape(1, 2)

    zpadrows = jnp.zeros((NPAD - N, 16), jnp.float32)

    def padded(ad16):
        return jnp.concatenate([ad16, zpadrows], axis=0)

    g1, ad1 = _tc_pre(x, W1, asf1, adf1, st4, 4, 128)
    a0, a1 = _sc_edge_128(srcp, dstp, g1, padded(ad1))
    h1, g2, ad2 = _combine1(a0[:N], a1[:N], g1, ad1, b1r, s4, None,
                            W2, asf2, adf2, st4, 4, 128)
    a0, a1 = _sc_edge_128(srcp, dstp, g2, padded(ad2))
    _, g3, ad3 = _combine2(a0[:N], a1[:N], g2, ad2, b2r, s4, h1,
                           W3, asf3, adf3, st1, 1, 32)
    a0, a1 = _sc_edge_32(srcp, dstp, g3, padded(ad3))
    rc = _combine_final(a0[:N], a1[:N], g3, ad3, b3r, s1, wrc, brc)
    return (rc[:, 0], rc[:, 1])


# speedup vs baseline: 1.0226x; 1.0226x over previous
import functools

import jax
import jax.numpy as jnp
from jax import lax
from jax.experimental import pallas as pl
from jax.experimental.pallas import tpu as pltpu
from jax.experimental.pallas import tpu_sc as plsc

N = 10000
E = 640000
NPAD = 10112
NC, NS = 2, 16
NW = NC * NS
EB = 112
BLK_C0 = 220
BLK_C1 = 140
NBLK = NS * (BLK_C0 + BLK_C1)
EPAD = NBLK * EB
RBLK = 1000
GRID = N // RBLK


def _make_sc_edge_kernel(RW, H):
    RWE = RW + 16
    rows_per_tile = NPAD // NS
    nvec = EB // 16
    mesh = plsc.VectorSubcoreMesh(core_axis_name="c", subcore_axis_name="s")

    @functools.partial(
        pl.kernel,
        out_type=(
            jax.ShapeDtypeStruct((NPAD, RWE), jnp.float32),
            jax.ShapeDtypeStruct((NPAD, RWE), jnp.float32),
        ),
        mesh=mesh,
        compiler_params=pltpu.CompilerParams(
            needs_layout_passes=False, use_tc_tiling_on_sc=False),
        scratch_types=[
            pltpu.VMEM((2, EB), jnp.int32),
            pltpu.VMEM((4, EB), jnp.int32),
            pltpu.VMEM((EB, RWE), jnp.float32),
            pltpu.VMEM((EB, RWE), jnp.float32),
            pltpu.VMEM((2, EB, 16), jnp.float32),
            pltpu.VMEM_SHARED((NPAD, RWE), jnp.float32),
        ] + [pltpu.SemaphoreType.DMA] * 8,
    )
    def sc_kernel(src_hbm, dst_hbm, g_hbm, ad_hbm,
                  acc0, acc1,
                  sbuf, dbuf, rows0, rows1, adb, acc_sp,
                  is0, is1, id0, id1, gs0, gs1, ss0, ss1):
        cid = lax.axis_index("c")
        sid = lax.axis_index("s")
        nb = jnp.where(cid == 0, BLK_C0, BLK_C1)
        rows = (rows0, rows1)
        isems = (is0, is1)
        idsems = (id0, id1)
        gsems = (gs0, gs1)
        ssems = (ss0, ss1)

        zero16 = jnp.zeros((16,), jnp.float32)

        @pl.loop(0, EB)
        def _zero_rows(e):
            for k in range(RWE // 16):
                rows0[e, pl.ds(k * 16, 16)] = zero16

        base = sid * rows_per_tile
        nfull = rows_per_tile // EB
        rem = rows_per_tile - nfull * EB
        for ch in range(nfull):
            pltpu.sync_copy(rows0, acc_sp.at[pl.ds(base + ch * EB, EB)])
        if rem:
            pltpu.sync_copy(rows0.at[pl.ds(0, rem)],
                            acc_sp.at[pl.ds(base + nfull * EB, rem)])
        plsc.subcore_barrier()

        iot = lax.iota(jnp.int32, 16)
        base_row = jnp.where(cid == 0, sid * BLK_C0,
                             NS * BLK_C0 + sid * BLK_C1)

        def compute_block(rws, adbj):
            @plsc.parallel_loop(0, EB, step=16, unroll=nvec)
            def _fphase(v16):
                ev16 = v16 + iot
                for h in range(H):
                    asv = plsc.load_gather(
                        rws, [ev16, jnp.full((16,), RW + h, jnp.int32)])
                    adv = plsc.load_gather(
                        adbj, [ev16, jnp.full((16,), h, jnp.int32)])
                    ev = asv + adv
                    ev = jnp.where(ev > 0, ev, ev * 0.2)
                    fv = jnp.exp(ev)
                    plsc.store_scatter(
                        rws, [ev16, jnp.full((16,), RW + h, jnp.int32)], fv)

            @plsc.parallel_loop(0, EB, unroll=4)
            def _scale(e):
                erow = jnp.full((16,), e, jnp.int32)
                for h in range(H):
                    fvec = plsc.load_gather(
                        rws, [erow, jnp.full((16,), RW + h, jnp.int32)])
                    for k in range(2):
                        col = h * 32 + k * 16
                        rws[e, pl.ds(col, 16)] = rws[e, pl.ds(col, 16)] * fvec

        pltpu.sync_copy(src_hbm.at[base_row], sbuf.at[0])
        pltpu.sync_copy(dst_hbm.at[base_row], dbuf.at[0])
        pltpu.async_copy(g_hbm.at[sbuf.at[0]], rows0, gs0)
        pltpu.async_copy(ad_hbm.at[dbuf.at[0]], adb.at[0], is0)

        @pl.loop(0, nb, step=4)
        def _outer(b0):
            for u in range(4):
                b = b0 + u
                j = u % 2
                nj = (u + 1) % 2
                nd = (u + 1) % 4

                @pl.when(b + 1 < nb)
                def _issue_idx():
                    pltpu.async_copy(src_hbm.at[base_row + b + 1],
                                     sbuf.at[nj], isems[nj])
                    pltpu.async_copy(dst_hbm.at[base_row + b + 1],
                                     dbuf.at[nd], idsems[nj])

                @pl.when(b >= 1)
                def _wait_scatter():
                    pltpu.make_async_copy(
                        rows[nj], acc_sp.at[dbuf.at[(u + 3) % 4]],
                        ssems[nj]).wait()

                @pl.when(b + 1 < nb)
                def _issue_gather():
                    pltpu.make_async_copy(src_hbm.at[base_row + b + 1],
                                          sbuf.at[nj], isems[nj]).wait()
                    pltpu.make_async_copy(dst_hbm.at[base_row + b + 1],
                                          dbuf.at[nd], idsems[nj]).wait()
                    pltpu.async_copy(g_hbm.at[sbuf.at[nj]], rows[nj],
                                     gsems[nj])
                    pltpu.async_copy(ad_hbm.at[dbuf.at[nd]], adb.at[nj],
                                     isems[nj])

                pltpu.make_async_copy(g_hbm.at[sbuf.at[j]], rows[j],
                                      gsems[j]).wait()
                pltpu.make_async_copy(ad_hbm.at[dbuf.at[u]], adb.at[j],
                                      isems[j]).wait()
                compute_block(rows[j], adb.at[j])
                pltpu.async_copy(rows[j], acc_sp.at[dbuf.at[u]], ssems[j],
                                 add=True)

        pltpu.make_async_copy(rows[1], acc_sp.at[dbuf.at[3]], ssems[1]).wait()
        plsc.subcore_barrier()

        @pl.when(cid == 0)
        def _out0():
            pltpu.sync_copy(acc_sp.at[pl.ds(base, rows_per_tile)],
                            acc0.at[pl.ds(base, rows_per_tile)])

        @pl.when(cid == 1)
        def _out1():
            pltpu.sync_copy(acc_sp.at[pl.ds(base, rows_per_tile)],
                            acc1.at[pl.ds(base, rows_per_tile)])

    return sc_kernel


_sc_edge_128 = _make_sc_edge_kernel(128, 4)
_sc_edge_32 = _make_sc_edge_kernel(32, 1)


def _full(shape):
    return pl.BlockSpec(shape, lambda i: (0,) * len(shape))


def _rows(width):
    return pl.BlockSpec((RBLK, width), lambda i: (i, 0))


def _logit_tables(xw, asf, adf, st, H):
    asn = jnp.dot(xw * asf, st, preferred_element_type=jnp.float32,
                  precision=lax.Precision.HIGHEST)
    adn = jnp.dot(xw * adf, st, preferred_element_type=jnp.float32,
                  precision=lax.Precision.HIGHEST)
    z = jnp.zeros((xw.shape[0], 16 - H), jnp.float32)
    g = jnp.concatenate([xw, asn, z], axis=1)
    ad16 = jnp.concatenate([adn, z], axis=1)
    return g, ad16


def _tc_pre_body(H, x_ref, w_ref, asf_ref, adf_ref, st_ref, g_ref, ad_ref):
    xw = jnp.dot(x_ref[...], w_ref[...], preferred_element_type=jnp.float32)
    g, ad16 = _logit_tables(xw, asf_ref[...], adf_ref[...], st_ref[...], H)
    g_ref[...] = g
    ad_ref[...] = ad16


def _tc_pre(x, w, asf, adf, st, H, RW):
    return pl.pallas_call(
        functools.partial(_tc_pre_body, H),
        grid=(GRID,),
        in_specs=[_rows(x.shape[1]), _full(w.shape), _full(asf.shape),
                  _full(adf.shape), _full(st.shape)],
        out_specs=[_rows(RW + 16), _rows(16)],
        out_shape=[
            jax.ShapeDtypeStruct((N, RW + 16), jnp.float32),
            jax.ShapeDtypeStruct((N, 16), jnp.float32),
        ],
    )(x, w, asf, adf, st)


def _elu(g):
    return jnp.where(g > 0, g, jnp.exp(g) - 1.0)


def _gat_epilogue(acc0, acc1, g, ad16, br, s, H, RW):
    xw = g[:, :RW]
    ev = g[:, RW:RW + H] + ad16[:, :H]
    fs = jnp.exp(jnp.where(ev > 0, ev, 0.2 * ev))
    dtot = acc0[:, RW:RW + H] + acc1[:, RW:RW + H] + fs
    num = acc0[:, :RW] + acc1[:, :RW] + jnp.dot(
        fs, s, preferred_element_type=jnp.float32,
        precision=lax.Precision.HIGHEST) * xw
    den = jnp.dot(dtot, s, preferred_element_type=jnp.float32,
                  precision=lax.Precision.HIGHEST) + 1e-16
    return _elu(num / den + br)


def _make_combine_mid(H, RW, residual):
    def body(*refs):
        if residual:
            (acc0, acc1, gr, adr, br, sr, resr,
             wn, asfn, adfn, stn, HN, h_out, gn_out, adn_out) = refs
        else:
            (acc0, acc1, gr, adr, br, sr,
             wn, asfn, adfn, stn, HN, h_out, gn_out, adn_out) = refs
        h = _gat_epilogue(acc0[...], acc1[...], gr[...], adr[...],
                          br[...], sr[...], H, RW)
        if residual:
            h = h + resr[...]
        h_out[...] = h
        xwn = jnp.dot(h, wn[...], preferred_element_type=jnp.float32)
        gn, adn16 = _logit_tables(xwn, asfn[...], adfn[...], stn[...], HN)
        gn_out[...] = gn
        adn_out[...] = adn16

    def call(acc0, acc1, g, ad16, b, s, res, wn, asfn, adfn, stn, HN, RWN):
        ins = [acc0, acc1, g, ad16, b, s]
        specs = [_rows(RW + 16), _rows(RW + 16), _rows(RW + 16), _rows(16),
                 _full(b.shape), _full(s.shape)]
        if residual:
            ins.append(res)
            specs.append(_rows(RW))
        ins += [wn, asfn, adfn, stn]
        specs += [_full(wn.shape), _full(asfn.shape), _full(adfn.shape),
                  _full(stn.shape)]
        return pl.pallas_call(
            lambda *refs: body(*refs[:len(ins)], HN, *refs[len(ins):]),
            grid=(GRID,),
            in_specs=specs,
            out_specs=[_rows(RW), _rows(RWN + 16), _rows(16)],
            out_shape=[
                jax.ShapeDtypeStruct((N, RW), jnp.float32),
                jax.ShapeDtypeStruct((N, RWN + 16), jnp.float32),
                jax.ShapeDtypeStruct((N, 16), jnp.float32),
            ],
        )(*ins)

    return call


_combine1 = _make_combine_mid(4, 128, residual=False)
_combine2 = _make_combine_mid(4, 128, residual=True)


def _combine_final_body(acc0, acc1, gr, adr, br, s1, wrc, brc, rc_out):
    h = _gat_epilogue(acc0[...], acc1[...], gr[...], adr[...],
                      br[...], s1[...], 1, 32)
    rc_out[...] = jnp.dot(h, wrc[...],
                          preferred_element_type=jnp.float32) + brc[...]


def _combine_final(acc0, acc1, g, ad16, b, s1, wrc, brc):
    return pl.pallas_call(
        _combine_final_body,
        grid=(GRID,),
        in_specs=[_rows(48), _rows(48), _rows(48), _rows(16),
                  _full(b.shape), _full(s1.shape), _full(wrc.shape),
                  _full(brc.shape)],
        out_specs=_rows(2),
        out_shape=jax.ShapeDtypeStruct((N, 2), jnp.float32),
    )(acc0, acc1, g, ad16, b, s1, wrc, brc)


def kernel(x, edge_index, W1, a_src1, a_dst1, b1, W2, a_src2, a_dst2, b2,
           W3, a_src3, a_dst3, b3, Wr, br, Wc, bc):
    pad = EPAD - E
    srcp = jnp.concatenate(
        [edge_index[0], jnp.zeros((pad,), jnp.int32)]).reshape(-1, EB)
    dstp = jnp.concatenate(
        [edge_index[1], jnp.full((pad,), N, jnp.int32)]).reshape(-1, EB)

    s4 = jnp.repeat(jnp.eye(4, dtype=jnp.float32), 32, axis=1)
    st4 = s4.T
    s1 = jnp.ones((1, 32), jnp.float32)
    st1 = jnp.ones((32, 1), jnp.float32)

    asf1 = a_src1.reshape(1, 128)
    adf1 = a_dst1.reshape(1, 128)
    asf2 = a_src2.reshape(1, 128)
    adf2 = a_dst2.reshape(1, 128)
    asf3 = a_src3.reshape(1, 32)
    adf3 = a_dst3.reshape(1, 32)
    b1r = b1.reshape(1, 128)
    b2r = b2.reshape(1, 128)
    b3r = b3.reshape(1, 32)
    wrc = jnp.concatenate([Wr, Wc], axis=1)
    brc = jnp.concatenate([br, bc]).reshape(1, 2)

    zpadrows = jnp.zeros((NPAD - N, 16), jnp.float32)

    def padded(ad16):
        return jnp.concatenate([ad16, zpadrows], axis=0)

    g1, ad1 = _tc_pre(x, W1, asf1, adf1, st4, 4, 128)
    a0, a1 = _sc_edge_128(srcp, dstp, g1, padded(ad1))
    h1, g2, ad2 = _combine1(a0[:N], a1[:N], g1, ad1, b1r, s4, None,
                            W2, asf2, adf2, st4, 4, 128)
    a0, a1 = _sc_edge_128(srcp, dstp, g2, padded(ad2))
    _, g3, ad3 = _combine2(a0[:N], a1[:N], g2, ad2, b2r, s4, h1,
                           W3, asf3, adf3, st1, 1, 32)
    a0, a1 = _sc_edge_32(srcp, dstp, g3, padded(ad3))
    rc = _combine_final(a0[:N], a1[:N], g3, ad3, b3r, s1, wrc, brc)
    return (rc[:, 0], rc[:, 1])
